# baseline (speedup 1.0000x reference)
import functools

import jax
import jax.numpy as jnp
from jax import lax
from jax.experimental import pallas as pl
from jax.experimental.pallas import tpu as pltpu
from jax.experimental.pallas import tpu_sc as plsc

_NC = 2
_NS = 16
_L = 16
_IC = 128
_SP = 10


@functools.lru_cache(maxsize=None)
def _build(S, D):
    P = 2 * S
    stage = P // _NS
    slabs = S // (_NC * _NS)
    st = slabs - _SP
    span = S + st - 1
    chunks = -(-span // _IC)
    rows = chunks * _IC

    mesh = plsc.VectorSubcoreMesh(core_axis_name="c", subcore_axis_name="s")

    @functools.partial(
        pl.kernel,
        out_type=jax.ShapeDtypeStruct((S * S, D), jnp.float32),
        mesh=mesh,
        scratch_types=[
            pltpu.VMEM((chunks, _IC), jnp.int32),
            pltpu.VMEM((rows, D), jnp.float32),
            pltpu.VMEM((stage,), jnp.int32),
            pltpu.VMEM((stage, D), jnp.float32),
            pltpu.VMEM_SHARED((P, D), jnp.float32),
            pltpu.SemaphoreType.DMA,
            pltpu.SemaphoreType.DMA,
            pltpu.SemaphoreType.DMA,
        ],
    )
    def k(table, out, idx_g, buf_v, idx_s, sbuf, rev, sem_g, sem_s, sem_c):
        c = lax.axis_index("c")
        s = lax.axis_index("s")
        row0 = c * (S // _NC) + s * slabs

        top = row0 + (S + slabs - 2)
        for j in range(chunks):
            idx_row = idx_g.at[j]
            for b in range(_IC // _L):
                v = (top - (j * _IC + b * _L)) - lax.iota(jnp.int32, _L)
                idx_row[pl.ds(b * _L, _L)] = jnp.maximum(v, 0)
        gathers = [
            pltpu.async_copy(table.at[idx_g.at[j]],
                             buf_v.at[pl.ds(j * _IC, _IC)], sem_g)
            for j in range(chunks)
        ]

        base = s * stage
        for b in range(stage // _L):
            v = (2 * S - 2) - base - (b * _L) - lax.iota(jnp.int32, _L)
            idx_s[pl.ds(b * _L, _L)] = jnp.maximum(v, 0)
        pltpu.async_copy(table.at[idx_s], sbuf, sem_s).wait()
        pltpu.sync_copy(sbuf, rev.at[pl.ds(base, stage)])

        for g in gathers:
            g.wait()
        copies = [
            pltpu.async_copy(buf_v.at[pl.ds(slabs - 1 - r, S)],
                             out.at[pl.ds((row0 + r) * S, S)], sem_c)
            for r in range(_SP, slabs)
        ]

        plsc.subcore_barrier()

        copies += [
            pltpu.async_copy(rev.at[pl.ds((S - 1) - (row0 + r), S)],
                             out.at[pl.ds((row0 + r) * S, S)], sem_c)
            for r in range(_SP)
        ]
        for cp in copies:
            cp.wait()

    return k


def kernel(rel_pos_emb, seq_len):
    del seq_len
    T, D = rel_pos_emb.shape
    S = (T + 1) // 2
    out2d = _build(S, D)(rel_pos_emb)
    return out2d.reshape(S, S, D)
